# baseline (speedup 1.0000x reference)
import functools

import jax
import jax.numpy as jnp
from jax import lax
from jax.experimental import pallas as pl
from jax.experimental.pallas import tpu as pltpu


def _dblock_kernel(x_ref, w1_ref, w2_ref, w3_ref, o_ref, col_ref, acc_ref,
                   *, C, H, W):
    f32 = jnp.float32
    bf16 = jnp.bfloat16
    HW = H * W

    idx = lax.broadcasted_iota(jnp.int32, (1, HW), 1)
    row = idx // W
    col = idx - row * W

    def build_col(cur, d):
        for kh in range(3):
            dr = (kh - 1) * d
            for kw in range(3):
                dc = (kw - 1) * d
                t = kh * 3 + kw
                s = dr * W + dc
                if s == 0:
                    tap = cur.astype(bf16)
                else:
                    shifted = pltpu.roll(cur, shift=(-s) % HW, axis=1)
                    m = None
                    if dr < 0:
                        m = row >= -dr
                    elif dr > 0:
                        m = row < H - dr
                    if dc < 0:
                        mc = col >= -dc
                    elif dc > 0:
                        mc = col < W - dc
                    else:
                        mc = None
                    if m is None:
                        m = mc
                    elif mc is not None:
                        m = m & mc
                    tap = jnp.where(m, shifted.astype(bf16), jnp.zeros((), bf16))
                col_ref[t * C:(t + 1) * C, :] = tap

    def conv(w_ref):
        y = jnp.dot(w_ref[...], col_ref[...], preferred_element_type=f32)
        return jnp.maximum(y, 0.0)

    x = x_ref[0].reshape(C, HW)
    build_col(x, 1)
    d1 = conv(w1_ref)
    acc_ref[...] = x + d1
    build_col(d1, 2)
    d2 = conv(w2_ref)
    acc_ref[...] += d2
    build_col(d2, 4)
    d3 = conv(w3_ref)
    o_ref[0] = (acc_ref[...] + d3).reshape(C, H, W)


def _dblock(x_nchw, w1, w2, w3):
    B, C, H, W = x_nchw.shape
    HW = H * W
    ws = [jnp.transpose(w.reshape(9 * C, C)).astype(jnp.bfloat16)
          for w in (w1, w2, w3)]

    flops = 3 * 2 * HW * (9 * C) * C * B
    bytes_accessed = 2 * B * C * HW * 4 + 3 * 9 * C * C * 2
    out = pl.pallas_call(
        functools.partial(_dblock_kernel, C=C, H=H, W=W),
        out_shape=jax.ShapeDtypeStruct((B, C, H, W), x_nchw.dtype),
        grid=(B,),
        in_specs=[
            pl.BlockSpec((1, C, H, W), lambda b: (b, 0, 0, 0)),
            pl.BlockSpec(memory_space=pltpu.MemorySpace.VMEM),
            pl.BlockSpec(memory_space=pltpu.MemorySpace.VMEM),
            pl.BlockSpec(memory_space=pltpu.MemorySpace.VMEM),
        ],
        out_specs=pl.BlockSpec((1, C, H, W), lambda b: (b, 0, 0, 0)),
        scratch_shapes=[pltpu.VMEM((9 * C, HW), jnp.bfloat16),
                        pltpu.VMEM((C, HW), jnp.float32)],
        compiler_params=pltpu.CompilerParams(
            dimension_semantics=("parallel",)),
        cost_estimate=pl.CostEstimate(flops=flops, transcendentals=0,
                                      bytes_accessed=bytes_accessed),
    )(x_nchw, *ws)
    return out


def kernel(x_nchw, w1, w2, w3):
    return _dblock(x_nchw, w1, w2, w3)


# speedup vs baseline: 2.1807x; 2.1807x over previous
import functools

import jax
import jax.numpy as jnp
from jax import lax
from jax.experimental import pallas as pl
from jax.experimental.pallas import tpu as pltpu


def _dblock_kernel(x_ref, w1_ref, w2_ref, w3_ref, o_ref, col_ref, *, C, H, W):
    f32 = jnp.float32
    bf16 = jnp.bfloat16
    HW = H * W

    idx = lax.broadcasted_iota(jnp.int32, (1, HW), 1)
    row = idx // W
    col = idx - row * W

    def maski(dr, dc):
        m = None
        if dr < 0:
            m = row >= -dr
        elif dr > 0:
            m = row < H - dr
        if dc < 0:
            mc = col >= -dc
        elif dc > 0:
            mc = col < W - dc
        else:
            mc = None
        if m is None:
            m = mc
        elif mc is not None:
            m = m & mc
        return jnp.where(m, jnp.int32(-1), jnp.int32(0))

    def build_col(cur, d):
        cb = cur.astype(bf16)
        ci = pltpu.bitcast(cb, jnp.int32)
        for kh in range(3):
            dr = (kh - 1) * d
            for kw in range(3):
                dc = (kw - 1) * d
                t = kh * 3 + kw
                s = dr * W + dc
                if s == 0:
                    tap = cb
                else:
                    shifted = pltpu.roll(ci, shift=(-s) % HW, axis=1)
                    tap = pltpu.bitcast(shifted & maski(dr, dc), bf16)
                col_ref[t * C:(t + 1) * C, :] = tap

    def conv(w_ref):
        y = jnp.dot(w_ref[...], col_ref[...], preferred_element_type=f32)
        return jnp.maximum(y, 0.0)

    x = x_ref[0]
    build_col(x, 1)
    d1 = conv(w1_ref)
    o_ref[0] = x + d1
    build_col(d1, 2)
    d2 = conv(w2_ref)
    o_ref[0] += d2
    build_col(d2, 4)
    d3 = conv(w3_ref)
    o_ref[0] += d3


def _dblock(x_nchw, w1, w2, w3):
    B, C, H, W = x_nchw.shape
    HW = H * W
    x2 = x_nchw.reshape(B, C, HW)
    ws = [jnp.transpose(w.reshape(9 * C, C)).astype(jnp.bfloat16)
          for w in (w1, w2, w3)]

    flops = 3 * 2 * HW * (9 * C) * C * B
    bytes_accessed = 2 * B * C * HW * 4 + 3 * 9 * C * C * 2
    out = pl.pallas_call(
        functools.partial(_dblock_kernel, C=C, H=H, W=W),
        out_shape=jax.ShapeDtypeStruct((B, C, HW), x_nchw.dtype),
        grid=(B,),
        in_specs=[
            pl.BlockSpec((1, C, HW), lambda b: (b, 0, 0)),
            pl.BlockSpec(memory_space=pltpu.MemorySpace.VMEM),
            pl.BlockSpec(memory_space=pltpu.MemorySpace.VMEM),
            pl.BlockSpec(memory_space=pltpu.MemorySpace.VMEM),
        ],
        out_specs=pl.BlockSpec((1, C, HW), lambda b: (b, 0, 0)),
        scratch_shapes=[pltpu.VMEM((9 * C, HW), jnp.bfloat16)],
        compiler_params=pltpu.CompilerParams(
            dimension_semantics=("parallel",)),
        cost_estimate=pl.CostEstimate(flops=flops, transcendentals=0,
                                      bytes_accessed=bytes_accessed),
    )(x2, *ws)
    return out.reshape(B, C, H, W)


def kernel(x_nchw, w1, w2, w3):
    return _dblock(x_nchw, w1, w2, w3)


# speedup vs baseline: 2.4983x; 1.1457x over previous
import functools

import jax
import jax.numpy as jnp
from jax import lax
from jax.experimental import pallas as pl
from jax.experimental.pallas import tpu as pltpu


def _dblock_kernel(x_ref, w1_ref, w2_ref, w3_ref, o_ref, col_ref, *, C, H, W):
    f32 = jnp.float32
    bf16 = jnp.bfloat16
    HW = H * W

    idx = lax.broadcasted_iota(jnp.int32, (1, HW), 1)
    row = idx // W
    col = idx - row * W

    def maski(dr, dc):
        m = None
        if dr < 0:
            m = row >= -dr
        elif dr > 0:
            m = row < H - dr
        if dc < 0:
            mc = col >= -dc
        elif dc > 0:
            mc = col < W - dc
        else:
            mc = None
        if m is None:
            m = mc
        elif mc is not None:
            m = m & mc
        return jnp.where(m, jnp.int32(-1), jnp.int32(0))

    def build_col(cur, d):
        cb = cur.astype(bf16)
        ci = pltpu.bitcast(cb, jnp.int32)
        for kh in range(3):
            dr = (kh - 1) * d
            for kw in range(3):
                dc = (kw - 1) * d
                t = kh * 3 + kw
                s = dr * W + dc
                if s == 0:
                    tap = cb
                else:
                    shifted = pltpu.roll(ci, shift=(-s) % HW, axis=1)
                    tap = pltpu.bitcast(shifted & maski(dr, dc), bf16)
                col_ref[t * C:(t + 1) * C, :] = tap

    def conv(w_ref):
        y = jnp.dot(w_ref[...], col_ref[...], preferred_element_type=f32)
        return jnp.maximum(y, 0.0)

    G = x_ref.shape[0]
    for i in range(G):
        x = x_ref[i]
        build_col(x, 1)
        d1 = conv(w1_ref)
        o_ref[i] = x + d1
        build_col(d1, 2)
        d2 = conv(w2_ref)
        o_ref[i] += d2
        build_col(d2, 4)
        d3 = conv(w3_ref)
        o_ref[i] += d3


def _dblock(x_nchw, w1, w2, w3):
    B, C, H, W = x_nchw.shape
    HW = H * W
    x2 = x_nchw.reshape(B, C, HW)
    ws = [jnp.transpose(w.reshape(9 * C, C)).astype(jnp.bfloat16)
          for w in (w1, w2, w3)]

    flops = 3 * 2 * HW * (9 * C) * C * B
    bytes_accessed = 2 * B * C * HW * 4 + 3 * 9 * C * C * 2
    G = 4
    out = pl.pallas_call(
        functools.partial(_dblock_kernel, C=C, H=H, W=W),
        out_shape=jax.ShapeDtypeStruct((B, C, HW), x_nchw.dtype),
        grid=(B // G,),
        in_specs=[
            pl.BlockSpec((G, C, HW), lambda b: (b, 0, 0)),
            pl.BlockSpec(memory_space=pltpu.MemorySpace.VMEM),
            pl.BlockSpec(memory_space=pltpu.MemorySpace.VMEM),
            pl.BlockSpec(memory_space=pltpu.MemorySpace.VMEM),
        ],
        out_specs=pl.BlockSpec((G, C, HW), lambda b: (b, 0, 0)),
        scratch_shapes=[pltpu.VMEM((9 * C, HW), jnp.bfloat16)],
        compiler_params=pltpu.CompilerParams(
            dimension_semantics=("parallel",)),
        cost_estimate=pl.CostEstimate(flops=flops, transcendentals=0,
                                      bytes_accessed=bytes_accessed),
    )(x2, *ws)
    return out.reshape(B, C, H, W)


def kernel(x_nchw, w1, w2, w3):
    return _dblock(x_nchw, w1, w2, w3)


# speedup vs baseline: 2.5230x; 1.0099x over previous
import functools

import jax
import jax.numpy as jnp
from jax import lax
from jax.experimental import pallas as pl
from jax.experimental.pallas import tpu as pltpu


def _dblock_kernel(x_ref, w1_ref, w2_ref, w3_ref, o_ref, col_ref, *, C, H, W):
    f32 = jnp.float32
    bf16 = jnp.bfloat16
    HW = H * W

    idx = lax.broadcasted_iota(jnp.int32, (1, HW), 1)
    row = idx // W
    col = idx - row * W

    def maski(dr, dc):
        m = None
        if dr < 0:
            m = row >= -dr
        elif dr > 0:
            m = row < H - dr
        if dc < 0:
            mc = col >= -dc
        elif dc > 0:
            mc = col < W - dc
        else:
            mc = None
        if m is None:
            m = mc
        elif mc is not None:
            m = m & mc
        return jnp.where(m, jnp.int32(-1), jnp.int32(0))

    def build_col(cur, d):
        cb = cur.astype(bf16)
        ci = pltpu.bitcast(cb, jnp.int32)
        for kh in range(3):
            dr = (kh - 1) * d
            for kw in range(3):
                dc = (kw - 1) * d
                t = kh * 3 + kw
                s = dr * W + dc
                if s == 0:
                    tap = cb
                else:
                    shifted = pltpu.roll(ci, shift=(-s) % HW, axis=1)
                    tap = pltpu.bitcast(shifted & maski(dr, dc), bf16)
                col_ref[t * C:(t + 1) * C, :] = tap

    def conv(w_ref):
        y = jnp.dot(w_ref[...], col_ref[...], preferred_element_type=f32)
        return jnp.maximum(y, 0.0)

    G = x_ref.shape[0]
    for i in range(G):
        x = x_ref[i]
        build_col(x, 1)
        d1 = conv(w1_ref)
        o_ref[i] = x + d1
        build_col(d1, 2)
        d2 = conv(w2_ref)
        o_ref[i] += d2
        build_col(d2, 4)
        d3 = conv(w3_ref)
        o_ref[i] += d3


def _dblock(x_nchw, w1, w2, w3):
    B, C, H, W = x_nchw.shape
    HW = H * W
    x2 = x_nchw.reshape(B, C, HW)
    ws = [jnp.transpose(w.reshape(9 * C, C)).astype(jnp.bfloat16)
          for w in (w1, w2, w3)]

    flops = 3 * 2 * HW * (9 * C) * C * B
    bytes_accessed = 2 * B * C * HW * 4 + 3 * 9 * C * C * 2
    G = 8
    out = pl.pallas_call(
        functools.partial(_dblock_kernel, C=C, H=H, W=W),
        out_shape=jax.ShapeDtypeStruct((B, C, HW), x_nchw.dtype),
        grid=(B // G,),
        in_specs=[
            pl.BlockSpec((G, C, HW), lambda b: (b, 0, 0)),
            pl.BlockSpec(memory_space=pltpu.MemorySpace.VMEM),
            pl.BlockSpec(memory_space=pltpu.MemorySpace.VMEM),
            pl.BlockSpec(memory_space=pltpu.MemorySpace.VMEM),
        ],
        out_specs=pl.BlockSpec((G, C, HW), lambda b: (b, 0, 0)),
        scratch_shapes=[pltpu.VMEM((9 * C, HW), jnp.bfloat16)],
        compiler_params=pltpu.CompilerParams(
            dimension_semantics=("parallel",)),
        cost_estimate=pl.CostEstimate(flops=flops, transcendentals=0,
                                      bytes_accessed=bytes_accessed),
    )(x2, *ws)
    return out.reshape(B, C, H, W)


def kernel(x_nchw, w1, w2, w3):
    return _dblock(x_nchw, w1, w2, w3)
